# baseline (device time: 30264 ns/iter reference)
import jax
import jax.numpy as jnp
from jax import lax
from jax.experimental import pallas as pl
from jax.experimental.pallas import tpu as pltpu

N_DEV = 16
STAGES = 4


def kernel(x, Wg, Wu, Wd):
    m, k = x.shape
    h = Wg.shape[1]
    n = Wd.shape[1]

    def body(x_ref, wg_ref, wu_ref, wd_ref, out_ref,
             send_ref, recv_ref, send_sems, recv_sems):
        my = lax.axis_index("i")

        xb = x_ref[...].astype(jnp.bfloat16)
        gate = jnp.dot(xb, wg_ref[...].astype(jnp.bfloat16),
                       preferred_element_type=jnp.float32)
        up = jnp.dot(xb, wu_ref[...].astype(jnp.bfloat16),
                     preferred_element_type=jnp.float32)
        silu_up = up / (1.0 + jnp.exp(-up))
        hact = (gate * silu_up).astype(jnp.bfloat16)
        acc = jnp.dot(hact, wd_ref[...].astype(jnp.bfloat16),
                      preferred_element_type=jnp.float32)

        for s in range(STAGES):
            partner = my ^ (1 << s)
            send_ref[s] = acc.astype(jnp.bfloat16)
            rdma = pltpu.make_async_remote_copy(
                src_ref=send_ref.at[s],
                dst_ref=recv_ref.at[s],
                send_sem=send_sems.at[s],
                recv_sem=recv_sems.at[s],
                device_id=(partner,),
                device_id_type=pl.DeviceIdType.MESH,
            )
            rdma.start()
            rdma.wait()
            acc = acc + recv_ref[s].astype(jnp.float32)

        out_ref[...] = acc

    return pl.pallas_call(
        body,
        out_shape=jax.ShapeDtypeStruct((m, n), jnp.float32),
        in_specs=[pl.BlockSpec(memory_space=pltpu.VMEM)] * 4,
        out_specs=pl.BlockSpec(memory_space=pltpu.VMEM),
        scratch_shapes=[
            pltpu.VMEM((STAGES, m, n), jnp.bfloat16),
            pltpu.VMEM((STAGES, m, n), jnp.bfloat16),
            pltpu.SemaphoreType.DMA((STAGES,)),
            pltpu.SemaphoreType.DMA((STAGES,)),
        ],
    )(x, Wg, Wu, Wd)


# device time: 24238 ns/iter; 1.2486x vs baseline; 1.2486x over previous
import jax
import jax.numpy as jnp
from jax import lax
from jax.experimental import pallas as pl
from jax.experimental.pallas import tpu as pltpu

N_DEV = 16
STAGES = 4


def kernel(x, Wg, Wu, Wd):
    m, k = x.shape
    h = Wg.shape[1]
    n = Wd.shape[1]

    def body(x_ref, wg_ref, wu_ref, wd_ref, out_ref,
             send_ref, recv_ref, send_sems, recv_sems):
        my = lax.axis_index("i")
        partners = [my ^ (1 << s) for s in range(STAGES)]

        barrier_sem = pltpu.get_barrier_semaphore()
        for p in partners:
            pl.semaphore_signal(barrier_sem, inc=1, device_id=(p,),
                                device_id_type=pl.DeviceIdType.MESH)

        xb = x_ref[...].astype(jnp.bfloat16)
        gate = jnp.dot(xb, wg_ref[...].astype(jnp.bfloat16),
                       preferred_element_type=jnp.float32)
        up = jnp.dot(xb, wu_ref[...].astype(jnp.bfloat16),
                     preferred_element_type=jnp.float32)
        silu_up = up / (1.0 + jnp.exp(-up))
        hact = (gate * silu_up).astype(jnp.bfloat16)
        acc = jnp.dot(hact, wd_ref[...].astype(jnp.bfloat16),
                      preferred_element_type=jnp.float32)

        pl.semaphore_wait(barrier_sem, STAGES)

        rdmas = []
        for s in range(STAGES):
            send_ref[s] = acc.astype(jnp.bfloat16)
            rdma = pltpu.make_async_remote_copy(
                src_ref=send_ref.at[s],
                dst_ref=recv_ref.at[s],
                send_sem=send_sems.at[s],
                recv_sem=recv_sems.at[s],
                device_id=(partners[s],),
                device_id_type=pl.DeviceIdType.MESH,
            )
            rdma.start()
            rdmas.append(rdma)
            rdma.wait_recv()
            acc = acc + recv_ref[s].astype(jnp.float32)

        out_ref[...] = acc
        for rdma in rdmas:
            rdma.wait_send()

    return pl.pallas_call(
        body,
        out_shape=jax.ShapeDtypeStruct((m, n), jnp.float32),
        in_specs=[pl.BlockSpec(memory_space=pltpu.VMEM)] * 4,
        out_specs=pl.BlockSpec(memory_space=pltpu.VMEM),
        scratch_shapes=[
            pltpu.VMEM((STAGES, m, n), jnp.bfloat16),
            pltpu.VMEM((STAGES, m, n), jnp.bfloat16),
            pltpu.SemaphoreType.DMA((STAGES,)),
            pltpu.SemaphoreType.DMA((STAGES,)),
        ],
        compiler_params=pltpu.CompilerParams(collective_id=0),
    )(x, Wg, Wu, Wd)


# device time: 20042 ns/iter; 1.5100x vs baseline; 1.2094x over previous
import jax
import jax.numpy as jnp
from jax import lax
from jax.experimental import pallas as pl
from jax.experimental.pallas import tpu as pltpu

N_DEV = 16


def kernel(x, Wg, Wu, Wd):
    m, k = x.shape
    n = Wd.shape[1]
    ch = m // N_DEV

    def body(x_ref, wg_ref, wu_ref, wd_ref, out_ref,
             sc_send_ref, rs_recv_ref, ag_send_ref, ag_recv_ref,
             rs_send_sems, rs_recv_sems, ag_send_sems, ag_recv_sems):
        my = lax.axis_index("i")

        barrier_sem = pltpu.get_barrier_semaphore()
        for p in range(N_DEV):
            @pl.when(my != p)
            def _():
                pl.semaphore_signal(barrier_sem, inc=1, device_id=(p,),
                                    device_id_type=pl.DeviceIdType.MESH)

        xb = x_ref[...].astype(jnp.bfloat16)
        gate = jnp.dot(xb, wg_ref[...].astype(jnp.bfloat16),
                       preferred_element_type=jnp.float32)
        up = jnp.dot(xb, wu_ref[...].astype(jnp.bfloat16),
                     preferred_element_type=jnp.float32)
        silu_up = up / (1.0 + jnp.exp(-up))
        hact = (gate * silu_up).astype(jnp.bfloat16)
        acc = jnp.dot(hact, wd_ref[...].astype(jnp.bfloat16),
                      preferred_element_type=jnp.float32)
        pbf = acc.astype(jnp.bfloat16)
        sc_send_ref[...] = pbf

        pl.semaphore_wait(barrier_sem, N_DEV - 1)

        rs_rdmas = []
        for c in range(N_DEV):
            @pl.when(my != c)
            def _(c=c):
                rdma = pltpu.make_async_remote_copy(
                    src_ref=sc_send_ref.at[pl.ds(c * ch, ch), :],
                    dst_ref=rs_recv_ref.at[my],
                    send_sem=rs_send_sems.at[c],
                    recv_sem=rs_recv_sems.at[my],
                    device_id=(c,),
                    device_id_type=pl.DeviceIdType.MESH,
                )
                rdma.start()

        rs_recv_ref[my] = sc_send_ref[pl.ds(my * ch, ch), :]

        for s in range(N_DEV):
            @pl.when(my != s)
            def _(s=s):
                rdma = pltpu.make_async_remote_copy(
                    src_ref=sc_send_ref.at[pl.ds(0, ch), :],
                    dst_ref=rs_recv_ref.at[s],
                    send_sem=rs_send_sems.at[s],
                    recv_sem=rs_recv_sems.at[s],
                    device_id=(s,),
                    device_id_type=pl.DeviceIdType.MESH,
                )
                rdma.wait_recv()

        reduced = jnp.sum(rs_recv_ref[...].astype(jnp.float32), axis=0)
        ag_send_ref[...] = reduced.astype(jnp.bfloat16)

        for c in range(N_DEV):
            @pl.when(my != c)
            def _(c=c):
                rdma = pltpu.make_async_remote_copy(
                    src_ref=ag_send_ref,
                    dst_ref=ag_recv_ref.at[my],
                    send_sem=ag_send_sems.at[c],
                    recv_sem=ag_recv_sems.at[my],
                    device_id=(c,),
                    device_id_type=pl.DeviceIdType.MESH,
                )
                rdma.start()

        ag_recv_ref[my] = ag_send_ref[...]

        for s in range(N_DEV):
            @pl.when(my != s)
            def _(s=s):
                rdma = pltpu.make_async_remote_copy(
                    src_ref=ag_send_ref,
                    dst_ref=ag_recv_ref.at[s],
                    send_sem=ag_send_sems.at[s],
                    recv_sem=ag_recv_sems.at[s],
                    device_id=(s,),
                    device_id_type=pl.DeviceIdType.MESH,
                )
                rdma.wait_recv()

        out_ref[...] = ag_recv_ref[...].reshape(m, n).astype(jnp.float32)

        for c in range(N_DEV):
            @pl.when(my != c)
            def _(c=c):
                for sems in (rs_send_sems, ag_send_sems):
                    rdma = pltpu.make_async_remote_copy(
                        src_ref=sc_send_ref.at[pl.ds(0, ch), :],
                        dst_ref=rs_recv_ref.at[c],
                        send_sem=sems.at[c],
                        recv_sem=rs_recv_sems.at[c],
                        device_id=(c,),
                        device_id_type=pl.DeviceIdType.MESH,
                    )
                    rdma.wait_send()

    return pl.pallas_call(
        body,
        out_shape=jax.ShapeDtypeStruct((m, n), jnp.float32),
        in_specs=[pl.BlockSpec(memory_space=pltpu.VMEM)] * 4,
        out_specs=pl.BlockSpec(memory_space=pltpu.VMEM),
        scratch_shapes=[
            pltpu.VMEM((m, n), jnp.bfloat16),
            pltpu.VMEM((N_DEV, ch, n), jnp.bfloat16),
            pltpu.VMEM((ch, n), jnp.bfloat16),
            pltpu.VMEM((N_DEV, ch, n), jnp.bfloat16),
            pltpu.SemaphoreType.DMA((N_DEV,)),
            pltpu.SemaphoreType.DMA((N_DEV,)),
            pltpu.SemaphoreType.DMA((N_DEV,)),
            pltpu.SemaphoreType.DMA((N_DEV,)),
        ],
        compiler_params=pltpu.CompilerParams(collective_id=0),
    )(x, Wg, Wu, Wd)


# device time: 16646 ns/iter; 1.8181x vs baseline; 1.2040x over previous
import jax
import jax.numpy as jnp
from jax import lax
from jax.experimental import pallas as pl
from jax.experimental.pallas import tpu as pltpu

N_DEV = 16
NEAR_FIRST = [1, 15, 4, 12, 8, 3, 13, 5, 11, 2, 14, 7, 9, 6, 10]
FAR_FIRST = NEAR_FIRST[::-1]


def kernel(x, Wg, Wu, Wd):
    m, k = x.shape
    n = Wd.shape[1]
    ch = m // N_DEV

    def body(x_ref, wg_ref, wu_ref, wd_ref, out_ref,
             sc_send_ref, rs_recv_ref, ag_send_ref, ag_recv_ref,
             rs_send_sems, rs_recv_sems, ag_send_sems, ag_recv_sems):
        my = lax.axis_index("i")

        barrier_sem = pltpu.get_barrier_semaphore()
        for o in FAR_FIRST:
            pl.semaphore_signal(barrier_sem, inc=1,
                                device_id=((my + o) % N_DEV,),
                                device_id_type=pl.DeviceIdType.MESH)

        xb = x_ref[...].astype(jnp.bfloat16)
        gate = jnp.dot(xb, wg_ref[...].astype(jnp.bfloat16),
                       preferred_element_type=jnp.float32)
        up = jnp.dot(xb, wu_ref[...].astype(jnp.bfloat16),
                     preferred_element_type=jnp.float32)
        silu_up = up / (1.0 + jnp.exp(-up))
        hact = (gate * silu_up).astype(jnp.bfloat16)
        acc = jnp.dot(hact, wd_ref[...].astype(jnp.bfloat16),
                      preferred_element_type=jnp.float32)
        sc_send_ref[...] = acc.astype(jnp.bfloat16)

        pl.semaphore_wait(barrier_sem, N_DEV - 1)

        for o in FAR_FIRST:
            c = (my + o) % N_DEV
            rdma = pltpu.make_async_remote_copy(
                src_ref=sc_send_ref.at[pl.ds(c * ch, ch), :],
                dst_ref=rs_recv_ref.at[N_DEV - o],
                send_sem=rs_send_sems.at[o],
                recv_sem=rs_recv_sems.at[N_DEV - o],
                device_id=(c,),
                device_id_type=pl.DeviceIdType.MESH,
            )
            rdma.start()

        red = sc_send_ref[pl.ds(my * ch, ch), :].astype(jnp.float32)
        for j in NEAR_FIRST:
            s = (my + j) % N_DEV
            rdma = pltpu.make_async_remote_copy(
                src_ref=sc_send_ref.at[pl.ds(0, ch), :],
                dst_ref=rs_recv_ref.at[j],
                send_sem=rs_send_sems.at[j],
                recv_sem=rs_recv_sems.at[j],
                device_id=(s,),
                device_id_type=pl.DeviceIdType.MESH,
            )
            rdma.wait_recv()
            red = red + rs_recv_ref[j].astype(jnp.float32)

        ag_send_ref[...] = red.astype(jnp.bfloat16)
        out_ref[pl.ds(my * ch, ch), :] = red

        for o in FAR_FIRST:
            t = (my + o) % N_DEV
            rdma = pltpu.make_async_remote_copy(
                src_ref=ag_send_ref,
                dst_ref=ag_recv_ref.at[N_DEV - o],
                send_sem=ag_send_sems.at[o],
                recv_sem=ag_recv_sems.at[N_DEV - o],
                device_id=(t,),
                device_id_type=pl.DeviceIdType.MESH,
            )
            rdma.start()

        for j in NEAR_FIRST:
            ow = (my + j) % N_DEV
            rdma = pltpu.make_async_remote_copy(
                src_ref=ag_send_ref,
                dst_ref=ag_recv_ref.at[j],
                send_sem=ag_send_sems.at[j],
                recv_sem=ag_recv_sems.at[j],
                device_id=(ow,),
                device_id_type=pl.DeviceIdType.MESH,
            )
            rdma.wait_recv()
            out_ref[pl.ds(ow * ch, ch), :] = ag_recv_ref[j].astype(jnp.float32)

        for o in range(1, N_DEV):
            for sems in (rs_send_sems, ag_send_sems):
                rdma = pltpu.make_async_remote_copy(
                    src_ref=sc_send_ref.at[pl.ds(0, ch), :],
                    dst_ref=rs_recv_ref.at[o],
                    send_sem=sems.at[o],
                    recv_sem=rs_recv_sems.at[o],
                    device_id=((my + o) % N_DEV,),
                    device_id_type=pl.DeviceIdType.MESH,
                )
                rdma.wait_send()

    return pl.pallas_call(
        body,
        out_shape=jax.ShapeDtypeStruct((m, n), jnp.float32),
        in_specs=[pl.BlockSpec(memory_space=pltpu.VMEM)] * 4,
        out_specs=pl.BlockSpec(memory_space=pltpu.VMEM),
        scratch_shapes=[
            pltpu.VMEM((m, n), jnp.bfloat16),
            pltpu.VMEM((N_DEV, ch, n), jnp.bfloat16),
            pltpu.VMEM((ch, n), jnp.bfloat16),
            pltpu.VMEM((N_DEV, ch, n), jnp.bfloat16),
            pltpu.SemaphoreType.DMA((N_DEV,)),
            pltpu.SemaphoreType.DMA((N_DEV,)),
            pltpu.SemaphoreType.DMA((N_DEV,)),
            pltpu.SemaphoreType.DMA((N_DEV,)),
        ],
        compiler_params=pltpu.CompilerParams(collective_id=0),
    )(x, Wg, Wu, Wd)


# device time: 16580 ns/iter; 1.8253x vs baseline; 1.0040x over previous
import jax
import jax.numpy as jnp
from jax import lax
from jax.experimental import pallas as pl
from jax.experimental.pallas import tpu as pltpu

N_DEV = 16
NEAR_FIRST = [1, 15, 4, 12, 8, 3, 13, 5, 11, 2, 14, 7, 9, 6, 10]
FAR_FIRST = NEAR_FIRST[::-1]


def kernel(x, Wg, Wu, Wd):
    m, k = x.shape
    n = Wd.shape[1]
    ch = m // N_DEV

    def body(x_ref, wg_ref, wu_ref, wd_ref, out_ref,
             sc_send_ref, rs_recv_ref, ag_send_ref,
             rs_send_sems, rs_recv_sems, ag_send_sems, ag_recv_sems):
        my = lax.axis_index("i")

        barrier_sem = pltpu.get_barrier_semaphore()
        for o in FAR_FIRST:
            pl.semaphore_signal(barrier_sem, inc=1,
                                device_id=((my + o) % N_DEV,),
                                device_id_type=pl.DeviceIdType.MESH)

        xb = x_ref[...].astype(jnp.bfloat16)
        gate = jnp.dot(xb, wg_ref[...].astype(jnp.bfloat16),
                       preferred_element_type=jnp.float32)
        up = jnp.dot(xb, wu_ref[...].astype(jnp.bfloat16),
                     preferred_element_type=jnp.float32)
        silu_up = up / (1.0 + jnp.exp(-up))
        hact = (gate * silu_up).astype(jnp.bfloat16)
        acc = jnp.dot(hact, wd_ref[...].astype(jnp.bfloat16),
                      preferred_element_type=jnp.float32)
        sc_send_ref[...] = acc.astype(jnp.bfloat16)

        pl.semaphore_wait(barrier_sem, N_DEV - 1)

        for o in FAR_FIRST:
            c = (my + o) % N_DEV
            rdma = pltpu.make_async_remote_copy(
                src_ref=sc_send_ref.at[pl.ds(c * ch, ch), :],
                dst_ref=rs_recv_ref.at[N_DEV - o],
                send_sem=rs_send_sems.at[o],
                recv_sem=rs_recv_sems.at[N_DEV - o],
                device_id=(c,),
                device_id_type=pl.DeviceIdType.MESH,
            )
            rdma.start()

        red = sc_send_ref[pl.ds(my * ch, ch), :].astype(jnp.float32)
        for j in NEAR_FIRST:
            s = (my + j) % N_DEV
            rdma = pltpu.make_async_remote_copy(
                src_ref=sc_send_ref.at[pl.ds(0, ch), :],
                dst_ref=rs_recv_ref.at[j],
                send_sem=rs_send_sems.at[j],
                recv_sem=rs_recv_sems.at[j],
                device_id=(s,),
                device_id_type=pl.DeviceIdType.MESH,
            )
            rdma.wait_recv()
            red = red + rs_recv_ref[j].astype(jnp.float32)

        ag_send_ref[...] = red.astype(jnp.bfloat16)
        out_ref[pl.ds(my * ch, ch), :] = ag_send_ref[...]

        for o in FAR_FIRST:
            t = (my + o) % N_DEV
            rdma = pltpu.make_async_remote_copy(
                src_ref=ag_send_ref,
                dst_ref=out_ref.at[pl.ds(my * ch, ch), :],
                send_sem=ag_send_sems.at[o],
                recv_sem=ag_recv_sems.at[N_DEV - o],
                device_id=(t,),
                device_id_type=pl.DeviceIdType.MESH,
            )
            rdma.start()

        for j in NEAR_FIRST:
            ow = (my + j) % N_DEV
            rdma = pltpu.make_async_remote_copy(
                src_ref=ag_send_ref,
                dst_ref=out_ref.at[pl.ds(ow * ch, ch), :],
                send_sem=ag_send_sems.at[j],
                recv_sem=ag_recv_sems.at[j],
                device_id=(ow,),
                device_id_type=pl.DeviceIdType.MESH,
            )
            rdma.wait_recv()

        for o in range(1, N_DEV):
            for sems in (rs_send_sems, ag_send_sems):
                rdma = pltpu.make_async_remote_copy(
                    src_ref=sc_send_ref.at[pl.ds(0, ch), :],
                    dst_ref=rs_recv_ref.at[o],
                    send_sem=sems.at[o],
                    recv_sem=rs_recv_sems.at[o],
                    device_id=((my + o) % N_DEV,),
                    device_id_type=pl.DeviceIdType.MESH,
                )
                rdma.wait_send()

    return pl.pallas_call(
        body,
        out_shape=jax.ShapeDtypeStruct((m, n), jnp.bfloat16),
        in_specs=[pl.BlockSpec(memory_space=pltpu.VMEM)] * 4,
        out_specs=pl.BlockSpec(memory_space=pltpu.VMEM),
        scratch_shapes=[
            pltpu.VMEM((m, n), jnp.bfloat16),
            pltpu.VMEM((N_DEV, ch, n), jnp.bfloat16),
            pltpu.VMEM((ch, n), jnp.bfloat16),
            pltpu.SemaphoreType.DMA((N_DEV,)),
            pltpu.SemaphoreType.DMA((N_DEV,)),
            pltpu.SemaphoreType.DMA((N_DEV,)),
            pltpu.SemaphoreType.DMA((N_DEV,)),
        ],
        compiler_params=pltpu.CompilerParams(collective_id=0),
    )(x, Wg, Wu, Wd)
